# baseline (speedup 1.0000x reference)
import functools

import jax
import jax.numpy as jnp
from jax import lax
from jax.experimental import pallas as pl
from jax.experimental.pallas import tpu as pltpu
from jax.experimental.pallas import tpu_sc as plsc

N = 10000
D = 128
E = 320000
LBL = 200000
NPAD = 10240

NC = 2
NS = 16
LANES = 16

CHUNK = 400
E_CHUNKS = E // CHUNK
E_CHUNKS_PER_TILE = E_CHUNKS // (NC * NS)
L_CHUNKS = LBL // CHUNK

_mesh = plsc.VectorSubcoreMesh(core_axis_name="c", subcore_axis_name="s")


def _fill(ref, start, count, value):
    v = jnp.full((LANES,), value, ref.dtype)

    def body(i, _):
        ref[pl.ds(start + i * LANES, LANES)] = v
        return 0

    lax.fori_loop(0, count // LANES, body, 0)


@functools.partial(
    pl.kernel,
    out_type=jax.ShapeDtypeStruct((NPAD,), jnp.float32),
    mesh=_mesh,
    scratch_types=[
        pltpu.VMEM((CHUNK,), jnp.int32),
        pltpu.VMEM((CHUNK,), jnp.float32),
        pltpu.VMEM((NPAD // NS,), jnp.float32),
        pltpu.VMEM_SHARED((NPAD,), jnp.float32),
    ],
)
def _deg_kernel(dst_hbm, dinv_hbm, idx_v, ones_v, slice_v, deg_sh):
    cid = lax.axis_index("c")
    sid = lax.axis_index("s")
    per = NPAD // NS

    _fill(slice_v, 0, per, 0.0)
    pltpu.sync_copy(slice_v, deg_sh.at[pl.ds(sid * per, per)])
    _fill(ones_v, 0, CHUNK, 1.0)
    plsc.subcore_barrier()

    def chunk_body(t, _):
        c = sid * (E_CHUNKS // NS) + t
        pltpu.sync_copy(dst_hbm.at[pl.ds(c * CHUNK, CHUNK)], idx_v)
        pltpu.sync_copy(ones_v, deg_sh.at[idx_v], add=True)
        return 0

    lax.fori_loop(0, E_CHUNKS // NS, chunk_body, 0)
    plsc.subcore_barrier()

    @pl.when(cid == 0)
    def _():
        pltpu.sync_copy(deg_sh.at[pl.ds(sid * per, per)],
                        dinv_hbm.at[pl.ds(sid * per, per)])


DH = D // 2
ECHUNK = 400
E_CHUNKS2 = E // ECHUNK
CPT = E_CHUNKS2 // NS


@functools.partial(
    pl.kernel,
    out_type=jax.ShapeDtypeStruct((NC, NPAD, DH), jnp.float32),
    mesh=_mesh,
    scratch_types=[
        pltpu.VMEM((ECHUNK,), jnp.int32),
        pltpu.VMEM((ECHUNK,), jnp.int32),
        pltpu.VMEM((ECHUNK,), jnp.int32),
        pltpu.VMEM((ECHUNK,), jnp.int32),
        pltpu.VMEM((ECHUNK,), jnp.int32),
        pltpu.VMEM((ECHUNK,), jnp.int32),
        pltpu.VMEM((ECHUNK, DH), jnp.float32),
        pltpu.VMEM((ECHUNK, DH), jnp.float32),
        pltpu.VMEM((ECHUNK, DH), jnp.float32),
        pltpu.SemaphoreType.DMA,
        pltpu.SemaphoreType.DMA,
        pltpu.SemaphoreType.DMA,
        pltpu.SemaphoreType.DMA,
        pltpu.SemaphoreType.DMA,
        pltpu.SemaphoreType.DMA,
        pltpu.VMEM_SHARED((NPAD, DH), jnp.float32),
    ],
    compiler_params=pltpu.CompilerParams(use_tc_tiling_on_sc=False, needs_layout_passes=False),
)
def _agg_kernel(gflat_hbm, src2_hbm, dst_hbm, out_hbm,
                src_a, src_b, src_c, dst_a, dst_b, dst_c,
                rows_a, rows_b, rows_c, gs_a, gs_b, gs_c, ss_a, ss_b, ss_c,
                acc_sh):
    cid = lax.axis_index("c")
    sid = lax.axis_index("s")
    rpt = NPAD // NS

    pltpu.sync_copy(gflat_hbm.at[pl.ds(cid * NPAD + sid * rpt, rpt)],
                    acc_sh.at[pl.ds(sid * rpt, rpt)])
    plsc.subcore_barrier()

    bufs = [(src_a, dst_a, rows_a, gs_a, ss_a),
            (src_b, dst_b, rows_b, gs_b, ss_b),
            (src_c, dst_c, rows_c, gs_c, ss_c)]

    def issue_gather(t):
        sv, dv, rv, gs, ss = bufs[t % 3]
        c = sid * CPT + t
        pltpu.sync_copy(src2_hbm.at[cid, pl.ds(c * ECHUNK, ECHUNK)], sv)
        pltpu.sync_copy(dst_hbm.at[pl.ds(c * ECHUNK, ECHUNK)], dv)
        pltpu.async_copy(gflat_hbm.at[sv], rv, gs)

    issue_gather(0)
    issue_gather(1)
    for t in range(CPT):
        sv, dv, rv, gs, ss = bufs[t % 3]
        if t + 2 < CPT:
            if t - 1 >= 0:
                psv, pdv, prv, pgs, pss = bufs[(t - 1) % 3]
                pltpu.make_async_copy(prv, acc_sh.at[pdv], pss).wait()
            issue_gather(t + 2)
        pltpu.make_async_copy(gflat_hbm.at[sv], rv, gs).wait()
        pltpu.async_copy(rv, acc_sh.at[dv], ss, add=True)

    for t in range(max(CPT - 3, 0), CPT):
        sv, dv, rv, gs, ss = bufs[t % 3]
        pltpu.make_async_copy(rv, acc_sh.at[dv], ss).wait()

    plsc.subcore_barrier()
    pltpu.sync_copy(acc_sh.at[pl.ds(sid * rpt, rpt)],
                    out_hbm.at[cid, pl.ds(sid * rpt, rpt)])


LCHUNK = 160
L_CHUNKS2 = LBL // LCHUNK
L_ITER = (L_CHUNKS2 + NC * NS - 1) // (NC * NS)


@functools.partial(
    pl.kernel,
    out_type=jax.ShapeDtypeStruct((LBL,), jnp.float32),
    mesh=_mesh,
    scratch_types=[
        pltpu.VMEM((LCHUNK,), jnp.int32),
        pltpu.VMEM((LCHUNK,), jnp.int32),
        pltpu.VMEM((LCHUNK,), jnp.int32),
        pltpu.VMEM((LCHUNK,), jnp.int32),
        pltpu.VMEM((LCHUNK, D), jnp.float32),
        pltpu.VMEM((LCHUNK, D), jnp.float32),
        pltpu.VMEM((LCHUNK, D), jnp.float32),
        pltpu.VMEM((LCHUNK, D), jnp.float32),
        pltpu.VMEM((LCHUNK,), jnp.float32),
        pltpu.SemaphoreType.DMA,
        pltpu.SemaphoreType.DMA,
    ],
    compiler_params=pltpu.CompilerParams(needs_layout_passes=False),
)
def _decode_kernel(z_hbm, a_hbm, b_hbm, out_hbm,
                   a_va, b_va, a_vb, b_vb, za_va, zb_va, za_vb, zb_vb,
                   dot_v, sem_a, sem_b):
    cid = lax.axis_index("c")
    sid = lax.axis_index("s")
    wid = sid * NC + cid

    bufs = [(a_va, b_va, za_va, zb_va, sem_a), (a_vb, b_vb, za_vb, zb_vb, sem_b)]
    lane = lax.iota(jnp.int32, LANES)

    def issue(t, buf):
        av, bv, zav, zbv, sem = buf
        c = wid + t * (NC * NS)

        @pl.when(c < L_CHUNKS2)
        def _():
            pltpu.sync_copy(a_hbm.at[pl.ds(c * LCHUNK, LCHUNK)], av)
            pltpu.sync_copy(b_hbm.at[pl.ds(c * LCHUNK, LCHUNK)], bv)
            pltpu.async_copy(z_hbm.at[av], zav, sem)
            pltpu.async_copy(z_hbm.at[bv], zbv, sem)

    def process(t, buf):
        av, bv, zav, zbv, sem = buf
        c = wid + t * (NC * NS)

        @pl.when(c < L_CHUNKS2)
        def _():
            pltpu.make_async_copy(z_hbm.at[av], zav, sem).wait()
            pltpu.make_async_copy(z_hbm.at[bv], zbv, sem).wait()

            def group_body(gidx, _):
                base = gidx * LANES
                vec = jnp.zeros((LANES,), jnp.float32)
                for j in range(LANES):
                    r = base + j
                    acc = zav[r, pl.ds(0, LANES)] * zbv[r, pl.ds(0, LANES)]
                    for k in range(1, D // LANES):
                        acc = acc + (
                            zav[r, pl.ds(k * LANES, LANES)]
                            * zbv[r, pl.ds(k * LANES, LANES)]
                        )
                    vec = jnp.where(lane == j, jnp.sum(acc), vec)
                dot_v[pl.ds(base, LANES)] = vec
                return 0

            lax.fori_loop(0, LCHUNK // LANES, group_body, 0)
            pltpu.sync_copy(dot_v, out_hbm.at[pl.ds(c * LCHUNK, LCHUNK)])

    issue(0, bufs[0])

    def chunk_iter(t, _):
        @pl.when(t % 2 == 0)
        def _():
            @pl.when(t + 1 < L_ITER)
            def _():
                issue(t + 1, bufs[1])
            process(t, bufs[0])

        @pl.when(t % 2 == 1)
        def _():
            @pl.when(t + 1 < L_ITER)
            def _():
                issue(t + 1, bufs[0])
            process(t, bufs[1])

        return 0

    lax.fori_loop(0, L_ITER, chunk_iter, 0)


RB = 1280
GRID = NPAD // RB

_row = pl.BlockSpec((RB, D), lambda i: (i, 0))
_col = pl.BlockSpec((RB, 1), lambda i: (i, 0))
_wgt = pl.BlockSpec((D, D), lambda i: (0, 0))
_bias = pl.BlockSpec((1, D), lambda i: (0, 0))


def _tc_a_body(x_ref, deg_ref, w1_ref, wf1_ref, bf1_ref, wf2_ref, bf2_ref,
               g1_ref, xf_ref, dinv_ref):
    xb = x_ref[...]
    dinv = lax.rsqrt(deg_ref[...] + 1.0)
    dinv_ref[...] = dinv
    g1_ref[...] = jnp.dot(xb, w1_ref[...], preferred_element_type=jnp.float32) * dinv
    t = jnp.maximum(
        jnp.dot(xb, wf1_ref[...], preferred_element_type=jnp.float32) + bf1_ref[...],
        0.0,
    )
    xf_ref[...] = jnp.dot(t, wf2_ref[...], preferred_element_type=jnp.float32) + bf2_ref[...]


_tc_a = pl.pallas_call(
    _tc_a_body,
    grid=(GRID,),
    in_specs=[_row, _col, _wgt, _wgt, _bias, _wgt, _bias],
    out_specs=[_row, _row, _col],
    out_shape=[
        jax.ShapeDtypeStruct((NPAD, D), jnp.float32),
        jax.ShapeDtypeStruct((NPAD, D), jnp.float32),
        jax.ShapeDtypeStruct((NPAD, 1), jnp.float32),
    ],
)


def _tc_b_body(s_ref, dinv_ref, b1_ref, w2_ref, g2_ref):
    dinv = dinv_ref[...]
    pre = s_ref[...] * dinv + b1_ref[...]
    out1 = jnp.maximum(pre, 0.0)
    g2_ref[...] = jnp.dot(out1, w2_ref[...], preferred_element_type=jnp.float32) * dinv


_tc_b = pl.pallas_call(
    _tc_b_body,
    grid=(GRID,),
    in_specs=[_row, _col, _bias, _wgt],
    out_specs=_row,
    out_shape=jax.ShapeDtypeStruct((NPAD, D), jnp.float32),
)


def _tc_c_body(s_ref, dinv_ref, b2_ref, xf_ref, z_ref):
    pre = s_ref[...] * dinv_ref[...] + b2_ref[...]
    z_ref[...] = 0.5 * pre + 0.5 * xf_ref[...]


_tc_c = pl.pallas_call(
    _tc_c_body,
    grid=(GRID,),
    in_specs=[_row, _col, _bias, _row],
    out_specs=_row,
    out_shape=jax.ShapeDtypeStruct((NPAD, D), jnp.float32),
)


def kernel(x, edge_index, edge_label_index, W1, b1, W2, b2, Wf1, bf1, Wf2, bf2):
    ei = edge_index.astype(jnp.int32)
    eli = edge_label_index.astype(jnp.int32)
    src = ei[0]
    dst = ei[1]

    xp = jnp.pad(x, ((0, NPAD - N), (0, 0)))
    deg = _deg_kernel(dst).reshape(NPAD, 1)

    b1r = b1.reshape(1, D)
    b2r = b2.reshape(1, D)
    bf1r = bf1.reshape(1, D)
    bf2r = bf2.reshape(1, D)

    src2 = jnp.stack([src, src + NPAD])

    g1, xf, dinv = _tc_a(xp, deg, W1, Wf1, bf1r, Wf2, bf2r)
    g1f = jnp.concatenate([g1[:, :DH], g1[:, DH:]], axis=0)
    s1h = _agg_kernel(g1f, src2, dst)
    s1 = s1h.transpose(1, 0, 2).reshape(NPAD, D)
    g2 = _tc_b(s1, dinv, b1r, W2)
    g2f = jnp.concatenate([g2[:, :DH], g2[:, DH:]], axis=0)
    s2h = _agg_kernel(g2f, src2, dst)
    s2 = s2h.transpose(1, 0, 2).reshape(NPAD, D)
    z = _tc_c(s2, dinv, b2r, xf)
    out = _decode_kernel(z, eli[0], eli[1])
    return out


# speedup vs baseline: 1.0953x; 1.0953x over previous
import functools

import jax
import jax.numpy as jnp
from jax import lax
from jax.experimental import pallas as pl
from jax.experimental.pallas import tpu as pltpu
from jax.experimental.pallas import tpu_sc as plsc

N = 10000
D = 128
E = 320000
LBL = 200000
NPAD = 10240

NC = 2
NS = 16
LANES = 16

CHUNK = 400
E_CHUNKS = E // CHUNK
E_CHUNKS_PER_TILE = E_CHUNKS // (NC * NS)
L_CHUNKS = LBL // CHUNK

_mesh = plsc.VectorSubcoreMesh(core_axis_name="c", subcore_axis_name="s")


def _fill(ref, start, count, value):
    v = jnp.full((LANES,), value, ref.dtype)

    def body(i, _):
        ref[pl.ds(start + i * LANES, LANES)] = v
        return 0

    lax.fori_loop(0, count // LANES, body, 0)


@functools.partial(
    pl.kernel,
    out_type=jax.ShapeDtypeStruct((NPAD,), jnp.float32),
    mesh=_mesh,
    scratch_types=[
        pltpu.VMEM((CHUNK,), jnp.int32),
        pltpu.VMEM((CHUNK,), jnp.float32),
        pltpu.VMEM((NPAD // NS,), jnp.float32),
        pltpu.VMEM_SHARED((NPAD,), jnp.float32),
    ],
)
def _deg_kernel(dst_hbm, dinv_hbm, idx_v, ones_v, slice_v, deg_sh):
    cid = lax.axis_index("c")
    sid = lax.axis_index("s")
    per = NPAD // NS

    _fill(slice_v, 0, per, 0.0)
    pltpu.sync_copy(slice_v, deg_sh.at[pl.ds(sid * per, per)])
    _fill(ones_v, 0, CHUNK, 1.0)
    plsc.subcore_barrier()

    def chunk_body(t, _):
        c = sid * (E_CHUNKS // NS) + t
        pltpu.sync_copy(dst_hbm.at[pl.ds(c * CHUNK, CHUNK)], idx_v)
        pltpu.sync_copy(ones_v, deg_sh.at[idx_v], add=True)
        return 0

    lax.fori_loop(0, E_CHUNKS // NS, chunk_body, 0)
    plsc.subcore_barrier()

    @pl.when(cid == 0)
    def _():
        pltpu.sync_copy(deg_sh.at[pl.ds(sid * per, per)],
                        dinv_hbm.at[pl.ds(sid * per, per)])


DH = D // 2
ECHUNK = 400
E_CHUNKS2 = E // ECHUNK
CPT = E_CHUNKS2 // NS


@functools.partial(
    pl.kernel,
    out_type=jax.ShapeDtypeStruct((NC, NPAD, DH), jnp.float32),
    mesh=_mesh,
    scratch_types=[
        pltpu.VMEM((ECHUNK,), jnp.int32),
        pltpu.VMEM((ECHUNK,), jnp.int32),
        pltpu.VMEM((ECHUNK,), jnp.int32),
        pltpu.VMEM((ECHUNK,), jnp.int32),
        pltpu.VMEM((ECHUNK,), jnp.int32),
        pltpu.VMEM((ECHUNK,), jnp.int32),
        pltpu.VMEM((ECHUNK, DH), jnp.float32),
        pltpu.VMEM((ECHUNK, DH), jnp.float32),
        pltpu.VMEM((ECHUNK, DH), jnp.float32),
        pltpu.SemaphoreType.DMA,
        pltpu.SemaphoreType.DMA,
        pltpu.SemaphoreType.DMA,
        pltpu.SemaphoreType.DMA,
        pltpu.SemaphoreType.DMA,
        pltpu.SemaphoreType.DMA,
        pltpu.VMEM_SHARED((NPAD, DH), jnp.float32),
    ],
    compiler_params=pltpu.CompilerParams(use_tc_tiling_on_sc=False, needs_layout_passes=False),
)
def _agg_kernel(gflat_hbm, src2_hbm, dst_hbm, out_hbm,
                src_a, src_b, src_c, dst_a, dst_b, dst_c,
                rows_a, rows_b, rows_c, gs_a, gs_b, gs_c, ss_a, ss_b, ss_c,
                acc_sh):
    cid = lax.axis_index("c")
    sid = lax.axis_index("s")
    rpt = NPAD // NS

    pltpu.sync_copy(gflat_hbm.at[pl.ds(cid * NPAD + sid * rpt, rpt)],
                    acc_sh.at[pl.ds(sid * rpt, rpt)])
    plsc.subcore_barrier()

    bufs = [(src_a, dst_a, rows_a, gs_a, ss_a),
            (src_b, dst_b, rows_b, gs_b, ss_b),
            (src_c, dst_c, rows_c, gs_c, ss_c)]

    def issue_gather(t):
        sv, dv, rv, gs, ss = bufs[t % 3]
        c = sid * CPT + t
        pltpu.sync_copy(src2_hbm.at[cid, pl.ds(c * ECHUNK, ECHUNK)], sv)
        pltpu.sync_copy(dst_hbm.at[pl.ds(c * ECHUNK, ECHUNK)], dv)
        pltpu.async_copy(gflat_hbm.at[sv], rv, gs)

    issue_gather(0)
    issue_gather(1)
    for t in range(CPT):
        sv, dv, rv, gs, ss = bufs[t % 3]
        if t + 2 < CPT:
            if t - 1 >= 0:
                psv, pdv, prv, pgs, pss = bufs[(t - 1) % 3]
                pltpu.make_async_copy(prv, acc_sh.at[pdv], pss).wait()
            issue_gather(t + 2)
        pltpu.make_async_copy(gflat_hbm.at[sv], rv, gs).wait()
        pltpu.async_copy(rv, acc_sh.at[dv], ss, add=True)

    for t in range(max(CPT - 3, 0), CPT):
        sv, dv, rv, gs, ss = bufs[t % 3]
        pltpu.make_async_copy(rv, acc_sh.at[dv], ss).wait()

    plsc.subcore_barrier()
    pltpu.sync_copy(acc_sh.at[pl.ds(sid * rpt, rpt)],
                    out_hbm.at[cid, pl.ds(sid * rpt, rpt)])


LCHUNK = 160
L_CHUNKS2 = LBL // LCHUNK
L_ITER = (L_CHUNKS2 + NC * NS - 1) // (NC * NS)


@functools.partial(
    pl.kernel,
    out_type=jax.ShapeDtypeStruct((LBL,), jnp.float32),
    mesh=_mesh,
    scratch_types=[
        pltpu.VMEM((LCHUNK,), jnp.int32),
        pltpu.VMEM((LCHUNK,), jnp.int32),
        pltpu.VMEM((LCHUNK,), jnp.int32),
        pltpu.VMEM((LCHUNK,), jnp.int32),
        pltpu.VMEM((LCHUNK, D), jnp.float32),
        pltpu.VMEM((LCHUNK, D), jnp.float32),
        pltpu.VMEM((LCHUNK, D), jnp.float32),
        pltpu.VMEM((LCHUNK, D), jnp.float32),
        pltpu.VMEM((LCHUNK,), jnp.float32),
        pltpu.SemaphoreType.DMA,
        pltpu.SemaphoreType.DMA,
    ],
    compiler_params=pltpu.CompilerParams(needs_layout_passes=False),
)
def _decode_kernel(z_hbm, a_hbm, b_hbm, out_hbm,
                   a_va, b_va, a_vb, b_vb, za_va, zb_va, za_vb, zb_vb,
                   dot_v, sem_a, sem_b):
    cid = lax.axis_index("c")
    sid = lax.axis_index("s")
    wid = sid * NC + cid

    bufs = [(a_va, b_va, za_va, zb_va, sem_a), (a_vb, b_vb, za_vb, zb_vb, sem_b)]
    lane = lax.iota(jnp.int32, LANES)

    def issue(t, buf):
        av, bv, zav, zbv, sem = buf
        c = wid + t * (NC * NS)

        @pl.when(c < L_CHUNKS2)
        def _():
            pltpu.sync_copy(a_hbm.at[pl.ds(c * LCHUNK, LCHUNK)], av)
            pltpu.sync_copy(b_hbm.at[pl.ds(c * LCHUNK, LCHUNK)], bv)
            pltpu.async_copy(z_hbm.at[av], zav, sem)
            pltpu.async_copy(z_hbm.at[bv], zbv, sem)

    def process(t, buf):
        av, bv, zav, zbv, sem = buf
        c = wid + t * (NC * NS)

        @pl.when(c < L_CHUNKS2)
        def _():
            pltpu.make_async_copy(z_hbm.at[av], zav, sem).wait()
            pltpu.make_async_copy(z_hbm.at[bv], zbv, sem).wait()

            def group_body(gidx, _):
                base = gidx * LANES
                rows16 = base + lane
                acc = jnp.zeros((LANES,), jnp.float32)
                for k in range(D):
                    col16 = (lane + k) & (D - 1)
                    va = plsc.load_gather(zav, [rows16, col16])
                    vb = plsc.load_gather(zbv, [rows16, col16])
                    acc = acc + va * vb
                dot_v[pl.ds(base, LANES)] = acc
                return 0

            lax.fori_loop(0, LCHUNK // LANES, group_body, 0)
            pltpu.sync_copy(dot_v, out_hbm.at[pl.ds(c * LCHUNK, LCHUNK)])

    issue(0, bufs[0])

    def chunk_iter(t, _):
        @pl.when(t % 2 == 0)
        def _():
            @pl.when(t + 1 < L_ITER)
            def _():
                issue(t + 1, bufs[1])
            process(t, bufs[0])

        @pl.when(t % 2 == 1)
        def _():
            @pl.when(t + 1 < L_ITER)
            def _():
                issue(t + 1, bufs[0])
            process(t, bufs[1])

        return 0

    lax.fori_loop(0, L_ITER, chunk_iter, 0)


RB = 1280
GRID = NPAD // RB

_row = pl.BlockSpec((RB, D), lambda i: (i, 0))
_col = pl.BlockSpec((RB, 1), lambda i: (i, 0))
_wgt = pl.BlockSpec((D, D), lambda i: (0, 0))
_bias = pl.BlockSpec((1, D), lambda i: (0, 0))


def _tc_a_body(x_ref, deg_ref, w1_ref, wf1_ref, bf1_ref, wf2_ref, bf2_ref,
               g1_ref, xf_ref, dinv_ref):
    xb = x_ref[...]
    dinv = lax.rsqrt(deg_ref[...] + 1.0)
    dinv_ref[...] = dinv
    g1_ref[...] = jnp.dot(xb, w1_ref[...], preferred_element_type=jnp.float32) * dinv
    t = jnp.maximum(
        jnp.dot(xb, wf1_ref[...], preferred_element_type=jnp.float32) + bf1_ref[...],
        0.0,
    )
    xf_ref[...] = jnp.dot(t, wf2_ref[...], preferred_element_type=jnp.float32) + bf2_ref[...]


_tc_a = pl.pallas_call(
    _tc_a_body,
    grid=(GRID,),
    in_specs=[_row, _col, _wgt, _wgt, _bias, _wgt, _bias],
    out_specs=[_row, _row, _col],
    out_shape=[
        jax.ShapeDtypeStruct((NPAD, D), jnp.float32),
        jax.ShapeDtypeStruct((NPAD, D), jnp.float32),
        jax.ShapeDtypeStruct((NPAD, 1), jnp.float32),
    ],
)


def _tc_b_body(s_ref, dinv_ref, b1_ref, w2_ref, g2_ref):
    dinv = dinv_ref[...]
    pre = s_ref[...] * dinv + b1_ref[...]
    out1 = jnp.maximum(pre, 0.0)
    g2_ref[...] = jnp.dot(out1, w2_ref[...], preferred_element_type=jnp.float32) * dinv


_tc_b = pl.pallas_call(
    _tc_b_body,
    grid=(GRID,),
    in_specs=[_row, _col, _bias, _wgt],
    out_specs=_row,
    out_shape=jax.ShapeDtypeStruct((NPAD, D), jnp.float32),
)


def _tc_c_body(s_ref, dinv_ref, b2_ref, xf_ref, z_ref):
    pre = s_ref[...] * dinv_ref[...] + b2_ref[...]
    z_ref[...] = 0.5 * pre + 0.5 * xf_ref[...]


_tc_c = pl.pallas_call(
    _tc_c_body,
    grid=(GRID,),
    in_specs=[_row, _col, _bias, _row],
    out_specs=_row,
    out_shape=jax.ShapeDtypeStruct((NPAD, D), jnp.float32),
)


def kernel(x, edge_index, edge_label_index, W1, b1, W2, b2, Wf1, bf1, Wf2, bf2):
    ei = edge_index.astype(jnp.int32)
    eli = edge_label_index.astype(jnp.int32)
    src = ei[0]
    dst = ei[1]

    xp = jnp.pad(x, ((0, NPAD - N), (0, 0)))
    deg = _deg_kernel(dst).reshape(NPAD, 1)

    b1r = b1.reshape(1, D)
    b2r = b2.reshape(1, D)
    bf1r = bf1.reshape(1, D)
    bf2r = bf2.reshape(1, D)

    src2 = jnp.stack([src, src + NPAD])

    g1, xf, dinv = _tc_a(xp, deg, W1, Wf1, bf1r, Wf2, bf2r)
    g1f = jnp.concatenate([g1[:, :DH], g1[:, DH:]], axis=0)
    s1h = _agg_kernel(g1f, src2, dst)
    s1 = s1h.transpose(1, 0, 2).reshape(NPAD, D)
    g2 = _tc_b(s1, dinv, b1r, W2)
    g2f = jnp.concatenate([g2[:, :DH], g2[:, DH:]], axis=0)
    s2h = _agg_kernel(g2f, src2, dst)
    s2 = s2h.transpose(1, 0, 2).reshape(NPAD, D)
    z = _tc_c(s2, dinv, b2r, xf)
    out = _decode_kernel(z, eli[0], eli[1])
    return out


# speedup vs baseline: 1.2827x; 1.1711x over previous
import functools

import jax
import jax.numpy as jnp
from jax import lax
from jax.experimental import pallas as pl
from jax.experimental.pallas import tpu as pltpu
from jax.experimental.pallas import tpu_sc as plsc

N = 10000
D = 128
E = 320000
LBL = 200000
NPAD = 10240

NC = 2
NS = 16
LANES = 16

CHUNK = 400
E_CHUNKS = E // CHUNK
E_CHUNKS_PER_TILE = E_CHUNKS // (NC * NS)
L_CHUNKS = LBL // CHUNK

_mesh = plsc.VectorSubcoreMesh(core_axis_name="c", subcore_axis_name="s")


def _fill(ref, start, count, value):
    v = jnp.full((LANES,), value, ref.dtype)

    def body(i, _):
        ref[pl.ds(start + i * LANES, LANES)] = v
        return 0

    lax.fori_loop(0, count // LANES, body, 0)


@functools.partial(
    pl.kernel,
    out_type=jax.ShapeDtypeStruct((NPAD,), jnp.float32),
    mesh=_mesh,
    scratch_types=[
        pltpu.VMEM((CHUNK,), jnp.int32),
        pltpu.VMEM((CHUNK,), jnp.float32),
        pltpu.VMEM((NPAD // NS,), jnp.float32),
        pltpu.VMEM_SHARED((NPAD,), jnp.float32),
    ],
)
def _deg_kernel(dst_hbm, dinv_hbm, idx_v, ones_v, slice_v, deg_sh):
    cid = lax.axis_index("c")
    sid = lax.axis_index("s")
    per = NPAD // NS

    _fill(slice_v, 0, per, 0.0)
    pltpu.sync_copy(slice_v, deg_sh.at[pl.ds(sid * per, per)])
    _fill(ones_v, 0, CHUNK, 1.0)
    plsc.subcore_barrier()

    def chunk_body(t, _):
        c = sid * (E_CHUNKS // NS) + t
        pltpu.sync_copy(dst_hbm.at[pl.ds(c * CHUNK, CHUNK)], idx_v)
        pltpu.sync_copy(ones_v, deg_sh.at[idx_v], add=True)
        return 0

    lax.fori_loop(0, E_CHUNKS // NS, chunk_body, 0)
    plsc.subcore_barrier()

    @pl.when(cid == 0)
    def _():
        pltpu.sync_copy(deg_sh.at[pl.ds(sid * per, per)],
                        dinv_hbm.at[pl.ds(sid * per, per)])


DH = D // 2
ECHUNK = 200
E_CHUNKS2 = E // ECHUNK
CPT = E_CHUNKS2 // NS


@functools.partial(
    pl.kernel,
    out_type=jax.ShapeDtypeStruct((NC, NPAD, DH), jnp.float32),
    mesh=_mesh,
    scratch_types=[
        pltpu.VMEM((CPT, ECHUNK), jnp.int32),
        pltpu.VMEM((CPT, ECHUNK), jnp.int32),
        pltpu.VMEM((ECHUNK, DH), jnp.float32),
        pltpu.VMEM((ECHUNK, DH), jnp.float32),
        pltpu.VMEM((ECHUNK, DH), jnp.float32),
        pltpu.SemaphoreType.DMA,
        pltpu.SemaphoreType.DMA,
        pltpu.SemaphoreType.DMA,
        pltpu.SemaphoreType.DMA,
        pltpu.SemaphoreType.DMA,
        pltpu.SemaphoreType.DMA,
        pltpu.VMEM_SHARED((NPAD, DH), jnp.float32),
    ],
    compiler_params=pltpu.CompilerParams(use_tc_tiling_on_sc=False, needs_layout_passes=False),
)
def _agg_kernel(gflat_hbm, src3_hbm, dst3_hbm, out_hbm,
                src_big, dst_big,
                rows_a, rows_b, rows_c, gs_a, gs_b, gs_c, ss_a, ss_b, ss_c,
                acc_sh):
    cid = lax.axis_index("c")
    sid = lax.axis_index("s")
    rpt = NPAD // NS

    pltpu.sync_copy(src3_hbm.at[cid, pl.ds(sid * CPT, CPT)], src_big)
    pltpu.sync_copy(dst3_hbm.at[pl.ds(sid * CPT, CPT)], dst_big)

    pltpu.sync_copy(gflat_hbm.at[pl.ds(cid * NPAD + sid * rpt, rpt)],
                    acc_sh.at[pl.ds(sid * rpt, rpt)])
    plsc.subcore_barrier()

    bufs = [(rows_a, gs_a, ss_a), (rows_b, gs_b, ss_b), (rows_c, gs_c, ss_c)]

    def issue_gather(t):
        rv, gs, ss = bufs[t % 3]
        pltpu.async_copy(gflat_hbm.at[src_big.at[t]], rv, gs)

    issue_gather(0)
    issue_gather(1)
    for t in range(CPT):
        rv, gs, ss = bufs[t % 3]
        if t + 2 < CPT:
            if t - 1 >= 0:
                prv, pgs, pss = bufs[(t - 1) % 3]
                pltpu.make_async_copy(prv, acc_sh.at[dst_big.at[t - 1]], pss).wait()
            issue_gather(t + 2)
        pltpu.make_async_copy(gflat_hbm.at[src_big.at[t]], rv, gs).wait()
        pltpu.async_copy(rv, acc_sh.at[dst_big.at[t]], ss, add=True)

    for t in range(max(CPT - 3, 0), CPT):
        rv, gs, ss = bufs[t % 3]
        pltpu.make_async_copy(rv, acc_sh.at[dst_big.at[t]], ss).wait()

    plsc.subcore_barrier()
    pltpu.sync_copy(acc_sh.at[pl.ds(sid * rpt, rpt)],
                    out_hbm.at[cid, pl.ds(sid * rpt, rpt)])


LCHUNK = 160
L_CHUNKS2 = LBL // LCHUNK
L_ITER = (L_CHUNKS2 + NC * NS - 1) // (NC * NS)


@functools.partial(
    pl.kernel,
    out_type=jax.ShapeDtypeStruct((LBL,), jnp.float32),
    mesh=_mesh,
    scratch_types=[
        pltpu.VMEM((L_ITER, LCHUNK), jnp.int32),
        pltpu.VMEM((L_ITER, LCHUNK), jnp.int32),
        pltpu.VMEM((LCHUNK, D), jnp.float32),
        pltpu.VMEM((LCHUNK, D), jnp.float32),
        pltpu.VMEM((LCHUNK, D), jnp.float32),
        pltpu.VMEM((LCHUNK, D), jnp.float32),
        pltpu.VMEM((LCHUNK,), jnp.float32),
        pltpu.SemaphoreType.DMA,
        pltpu.SemaphoreType.DMA,
    ],
    compiler_params=pltpu.CompilerParams(use_tc_tiling_on_sc=False, needs_layout_passes=False),
)
def _decode_kernel(z_hbm, a3_hbm, b3_hbm, out_hbm,
                   a_big, b_big, za_va, zb_va, za_vb, zb_vb,
                   dot_v, sem_a, sem_b):
    cid = lax.axis_index("c")
    sid = lax.axis_index("s")
    wid = sid * NC + cid

    pltpu.sync_copy(a3_hbm.at[pl.ds(wid * L_ITER, L_ITER)], a_big)
    pltpu.sync_copy(b3_hbm.at[pl.ds(wid * L_ITER, L_ITER)], b_big)

    bufs = [(za_va, zb_va, sem_a), (za_vb, zb_vb, sem_b)]
    lane = lax.iota(jnp.int32, LANES)

    def issue(t, buf):
        zav, zbv, sem = buf
        c = wid * L_ITER + t

        @pl.when(c < L_CHUNKS2)
        def _():
            pltpu.async_copy(z_hbm.at[a_big.at[t]], zav, sem)
            pltpu.async_copy(z_hbm.at[b_big.at[t]], zbv, sem)

    def process(t, buf):
        zav, zbv, sem = buf
        c = wid * L_ITER + t

        @pl.when(c < L_CHUNKS2)
        def _():
            pltpu.make_async_copy(z_hbm.at[a_big.at[t]], zav, sem).wait()
            pltpu.make_async_copy(z_hbm.at[b_big.at[t]], zbv, sem).wait()

            def group_body(gidx, _):
                base = gidx * LANES
                rows16 = base + lane
                acc = jnp.zeros((LANES,), jnp.float32)
                for k in range(D):
                    col16 = (lane + k) & (D - 1)
                    va = plsc.load_gather(zav, [rows16, col16])
                    vb = plsc.load_gather(zbv, [rows16, col16])
                    acc = acc + va * vb
                dot_v[pl.ds(base, LANES)] = acc
                return 0

            lax.fori_loop(0, LCHUNK // LANES, group_body, 0)
            pltpu.sync_copy(dot_v, out_hbm.at[pl.ds(c * LCHUNK, LCHUNK)])

    issue(0, bufs[0])

    def chunk_iter(t, _):
        @pl.when(t % 2 == 0)
        def _():
            @pl.when(t + 1 < L_ITER)
            def _():
                issue(t + 1, bufs[1])
            process(t, bufs[0])

        @pl.when(t % 2 == 1)
        def _():
            @pl.when(t + 1 < L_ITER)
            def _():
                issue(t + 1, bufs[0])
            process(t, bufs[1])

        return 0

    lax.fori_loop(0, L_ITER, chunk_iter, 0)


RB = 1280
GRID = NPAD // RB

_row = pl.BlockSpec((RB, D), lambda i: (i, 0))
_col = pl.BlockSpec((RB, 1), lambda i: (i, 0))
_wgt = pl.BlockSpec((D, D), lambda i: (0, 0))
_bias = pl.BlockSpec((1, D), lambda i: (0, 0))


def _tc_a_body(x_ref, deg_ref, w1_ref, wf1_ref, bf1_ref, wf2_ref, bf2_ref,
               g1_ref, xf_ref, dinv_ref):
    xb = x_ref[...]
    dinv = lax.rsqrt(deg_ref[...] + 1.0)
    dinv_ref[...] = dinv
    g1_ref[...] = jnp.dot(xb, w1_ref[...], preferred_element_type=jnp.float32) * dinv
    t = jnp.maximum(
        jnp.dot(xb, wf1_ref[...], preferred_element_type=jnp.float32) + bf1_ref[...],
        0.0,
    )
    xf_ref[...] = jnp.dot(t, wf2_ref[...], preferred_element_type=jnp.float32) + bf2_ref[...]


_tc_a = pl.pallas_call(
    _tc_a_body,
    grid=(GRID,),
    in_specs=[_row, _col, _wgt, _wgt, _bias, _wgt, _bias],
    out_specs=[_row, _row, _col],
    out_shape=[
        jax.ShapeDtypeStruct((NPAD, D), jnp.float32),
        jax.ShapeDtypeStruct((NPAD, D), jnp.float32),
        jax.ShapeDtypeStruct((NPAD, 1), jnp.float32),
    ],
)


def _tc_b_body(s_ref, dinv_ref, b1_ref, w2_ref, g2_ref):
    dinv = dinv_ref[...]
    pre = s_ref[...] * dinv + b1_ref[...]
    out1 = jnp.maximum(pre, 0.0)
    g2_ref[...] = jnp.dot(out1, w2_ref[...], preferred_element_type=jnp.float32) * dinv


_tc_b = pl.pallas_call(
    _tc_b_body,
    grid=(GRID,),
    in_specs=[_row, _col, _bias, _wgt],
    out_specs=_row,
    out_shape=jax.ShapeDtypeStruct((NPAD, D), jnp.float32),
)


def _tc_c_body(s_ref, dinv_ref, b2_ref, xf_ref, z_ref):
    pre = s_ref[...] * dinv_ref[...] + b2_ref[...]
    z_ref[...] = 0.5 * pre + 0.5 * xf_ref[...]


_tc_c = pl.pallas_call(
    _tc_c_body,
    grid=(GRID,),
    in_specs=[_row, _col, _bias, _row],
    out_specs=_row,
    out_shape=jax.ShapeDtypeStruct((NPAD, D), jnp.float32),
)


def kernel(x, edge_index, edge_label_index, W1, b1, W2, b2, Wf1, bf1, Wf2, bf2):
    ei = edge_index.astype(jnp.int32)
    eli = edge_label_index.astype(jnp.int32)
    src = ei[0]
    dst = ei[1]

    xp = jnp.pad(x, ((0, NPAD - N), (0, 0)))
    deg = _deg_kernel(dst).reshape(NPAD, 1)

    b1r = b1.reshape(1, D)
    b2r = b2.reshape(1, D)
    bf1r = bf1.reshape(1, D)
    bf2r = bf2.reshape(1, D)

    src3 = jnp.stack([src, src + NPAD]).reshape(NC, E_CHUNKS2, ECHUNK)
    dst3 = dst.reshape(E_CHUNKS2, ECHUNK)

    g1, xf, dinv = _tc_a(xp, deg, W1, Wf1, bf1r, Wf2, bf2r)
    g1f = jnp.concatenate([g1[:, :DH], g1[:, DH:]], axis=0)
    s1h = _agg_kernel(g1f, src3, dst3)
    s1 = s1h.transpose(1, 0, 2).reshape(NPAD, D)
    g2 = _tc_b(s1, dinv, b1r, W2)
    g2f = jnp.concatenate([g2[:, :DH], g2[:, DH:]], axis=0)
    s2h = _agg_kernel(g2f, src3, dst3)
    s2 = s2h.transpose(1, 0, 2).reshape(NPAD, D)
    z = _tc_c(s2, dinv, b2r, xf)
    npad_lbl = NC * NS * L_ITER * LCHUNK - LBL
    ea3 = jnp.pad(eli[0], (0, npad_lbl)).reshape(NC * NS * L_ITER, LCHUNK)
    eb3 = jnp.pad(eli[1], (0, npad_lbl)).reshape(NC * NS * L_ITER, LCHUNK)
    out = _decode_kernel(z, ea3, eb3)
    return out


# speedup vs baseline: 1.3485x; 1.0513x over previous
import functools

import jax
import jax.numpy as jnp
from jax import lax
from jax.experimental import pallas as pl
from jax.experimental.pallas import tpu as pltpu
from jax.experimental.pallas import tpu_sc as plsc

N = 10000
D = 128
E = 320000
LBL = 200000
NPAD = 10240

NC = 2
NS = 16
LANES = 16

CHUNK = 400
E_CHUNKS = E // CHUNK
E_CHUNKS_PER_TILE = E_CHUNKS // (NC * NS)
L_CHUNKS = LBL // CHUNK

_mesh = plsc.VectorSubcoreMesh(core_axis_name="c", subcore_axis_name="s")


def _fill(ref, start, count, value):
    v = jnp.full((LANES,), value, ref.dtype)

    def body(i, _):
        ref[pl.ds(start + i * LANES, LANES)] = v
        return 0

    lax.fori_loop(0, count // LANES, body, 0)


DEG_CPT = (E // 200) // NS


@functools.partial(
    pl.kernel,
    out_type=jax.ShapeDtypeStruct((NPAD,), jnp.float32),
    mesh=_mesh,
    scratch_types=[
        pltpu.VMEM((DEG_CPT, 200), jnp.int32),
        pltpu.VMEM((200,), jnp.float32),
        pltpu.VMEM((NPAD // NS,), jnp.float32),
        pltpu.SemaphoreType.DMA,
        pltpu.VMEM_SHARED((NPAD,), jnp.float32),
    ],
    compiler_params=pltpu.CompilerParams(use_tc_tiling_on_sc=False, needs_layout_passes=False),
)
def _deg_kernel(dst3_hbm, deg_hbm, dst_big, ones_v, slice_v, ssem, deg_sh):
    cid = lax.axis_index("c")
    sid = lax.axis_index("s")
    per = NPAD // NS

    pltpu.sync_copy(dst3_hbm.at[pl.ds(sid * DEG_CPT, DEG_CPT)], dst_big)

    _fill(slice_v, 0, per, 0.0)
    pltpu.sync_copy(slice_v, deg_sh.at[pl.ds(sid * per, per)])
    _fill(ones_v, 0, 200, 1.0)
    plsc.subcore_barrier()

    for t in range(DEG_CPT):
        pltpu.async_copy(ones_v, deg_sh.at[dst_big.at[t]], ssem, add=True)
    for t in range(DEG_CPT):
        pltpu.make_async_copy(ones_v, deg_sh.at[dst_big.at[t]], ssem).wait()
    plsc.subcore_barrier()

    @pl.when(cid == 0)
    def _():
        pltpu.sync_copy(deg_sh.at[pl.ds(sid * per, per)],
                        deg_hbm.at[pl.ds(sid * per, per)])


DH = D // 2
ECHUNK = 200
E_CHUNKS2 = E // ECHUNK
CPT = E_CHUNKS2 // NS


@functools.partial(
    pl.kernel,
    out_type=jax.ShapeDtypeStruct((NC, NPAD, DH), jnp.float32),
    mesh=_mesh,
    scratch_types=[
        pltpu.VMEM((CPT, ECHUNK), jnp.int32),
        pltpu.VMEM((CPT, ECHUNK), jnp.int32),
        pltpu.VMEM((ECHUNK, DH), jnp.float32),
        pltpu.VMEM((ECHUNK, DH), jnp.float32),
        pltpu.VMEM((ECHUNK, DH), jnp.float32),
        pltpu.SemaphoreType.DMA,
        pltpu.SemaphoreType.DMA,
        pltpu.SemaphoreType.DMA,
        pltpu.SemaphoreType.DMA,
        pltpu.SemaphoreType.DMA,
        pltpu.SemaphoreType.DMA,
        pltpu.VMEM_SHARED((NPAD, DH), jnp.float32),
    ],
    compiler_params=pltpu.CompilerParams(use_tc_tiling_on_sc=False, needs_layout_passes=False),
)
def _agg_kernel(gflat_hbm, src3_hbm, dst3_hbm, out_hbm,
                src_big, dst_big,
                rows_a, rows_b, rows_c, gs_a, gs_b, gs_c, ss_a, ss_b, ss_c,
                acc_sh):
    cid = lax.axis_index("c")
    sid = lax.axis_index("s")
    rpt = NPAD // NS

    pltpu.sync_copy(src3_hbm.at[cid, pl.ds(sid * CPT, CPT)], src_big)
    pltpu.sync_copy(dst3_hbm.at[pl.ds(sid * CPT, CPT)], dst_big)

    pltpu.sync_copy(gflat_hbm.at[pl.ds(cid * NPAD + sid * rpt, rpt)],
                    acc_sh.at[pl.ds(sid * rpt, rpt)])
    plsc.subcore_barrier()

    bufs = [(rows_a, gs_a, ss_a), (rows_b, gs_b, ss_b), (rows_c, gs_c, ss_c)]

    def issue_gather(t):
        rv, gs, ss = bufs[t % 3]
        pltpu.async_copy(gflat_hbm.at[src_big.at[t]], rv, gs)

    issue_gather(0)
    issue_gather(1)
    for t in range(CPT):
        rv, gs, ss = bufs[t % 3]
        if t + 2 < CPT:
            if t - 1 >= 0:
                prv, pgs, pss = bufs[(t - 1) % 3]
                pltpu.make_async_copy(prv, acc_sh.at[dst_big.at[t - 1]], pss).wait()
            issue_gather(t + 2)
        pltpu.make_async_copy(gflat_hbm.at[src_big.at[t]], rv, gs).wait()
        pltpu.async_copy(rv, acc_sh.at[dst_big.at[t]], ss, add=True)

    for t in range(max(CPT - 3, 0), CPT):
        rv, gs, ss = bufs[t % 3]
        pltpu.make_async_copy(rv, acc_sh.at[dst_big.at[t]], ss).wait()

    plsc.subcore_barrier()
    pltpu.sync_copy(acc_sh.at[pl.ds(sid * rpt, rpt)],
                    out_hbm.at[cid, pl.ds(sid * rpt, rpt)])


LCHUNK = 160
L_CHUNKS2 = LBL // LCHUNK
L_ITER = (L_CHUNKS2 + NC * NS - 1) // (NC * NS)


@functools.partial(
    pl.kernel,
    out_type=jax.ShapeDtypeStruct((LBL,), jnp.float32),
    mesh=_mesh,
    scratch_types=[
        pltpu.VMEM((L_ITER, LCHUNK), jnp.int32),
        pltpu.VMEM((L_ITER, LCHUNK), jnp.int32),
        pltpu.VMEM((LCHUNK, D), jnp.float32),
        pltpu.VMEM((LCHUNK, D), jnp.float32),
        pltpu.VMEM((LCHUNK, D), jnp.float32),
        pltpu.VMEM((LCHUNK, D), jnp.float32),
        pltpu.VMEM((LCHUNK,), jnp.float32),
        pltpu.SemaphoreType.DMA,
        pltpu.SemaphoreType.DMA,
    ],
    compiler_params=pltpu.CompilerParams(use_tc_tiling_on_sc=False, needs_layout_passes=False),
)
def _decode_kernel(z_hbm, a3_hbm, b3_hbm, out_hbm,
                   a_big, b_big, za_va, zb_va, za_vb, zb_vb,
                   dot_v, sem_a, sem_b):
    cid = lax.axis_index("c")
    sid = lax.axis_index("s")
    wid = sid * NC + cid

    pltpu.sync_copy(a3_hbm.at[pl.ds(wid * L_ITER, L_ITER)], a_big)
    pltpu.sync_copy(b3_hbm.at[pl.ds(wid * L_ITER, L_ITER)], b_big)

    bufs = [(za_va, zb_va, sem_a), (za_vb, zb_vb, sem_b)]
    lane = lax.iota(jnp.int32, LANES)

    def issue(t, buf):
        zav, zbv, sem = buf
        c = wid * L_ITER + t

        @pl.when(c < L_CHUNKS2)
        def _():
            pltpu.async_copy(z_hbm.at[a_big.at[t]], zav, sem)
            pltpu.async_copy(z_hbm.at[b_big.at[t]], zbv, sem)

    def process(t, buf):
        zav, zbv, sem = buf
        c = wid * L_ITER + t

        @pl.when(c < L_CHUNKS2)
        def _():
            pltpu.make_async_copy(z_hbm.at[a_big.at[t]], zav, sem).wait()
            pltpu.make_async_copy(z_hbm.at[b_big.at[t]], zbv, sem).wait()

            def group_body(gidx, _):
                base = gidx * LANES
                rows16 = base + lane
                acc = jnp.zeros((LANES,), jnp.float32)
                for k in range(D):
                    col16 = (lane + k) & (D - 1)
                    va = plsc.load_gather(zav, [rows16, col16])
                    vb = plsc.load_gather(zbv, [rows16, col16])
                    acc = acc + va * vb
                dot_v[pl.ds(base, LANES)] = acc
                return 0

            lax.fori_loop(0, LCHUNK // LANES, group_body, 0)
            pltpu.sync_copy(dot_v, out_hbm.at[pl.ds(c * LCHUNK, LCHUNK)])

    issue(0, bufs[0])

    def chunk_iter(t, _):
        @pl.when(t % 2 == 0)
        def _():
            @pl.when(t + 1 < L_ITER)
            def _():
                issue(t + 1, bufs[1])
            process(t, bufs[0])

        @pl.when(t % 2 == 1)
        def _():
            @pl.when(t + 1 < L_ITER)
            def _():
                issue(t + 1, bufs[0])
            process(t, bufs[1])

        return 0

    lax.fori_loop(0, L_ITER, chunk_iter, 0)


RB = 1280
GRID = NPAD // RB

_row = pl.BlockSpec((RB, D), lambda i: (i, 0))
_col = pl.BlockSpec((RB, 1), lambda i: (i, 0))
_wgt = pl.BlockSpec((D, D), lambda i: (0, 0))
_bias = pl.BlockSpec((1, D), lambda i: (0, 0))


def _tc_a_body(x_ref, deg_ref, w1_ref, wf1_ref, bf1_ref, wf2_ref, bf2_ref,
               g1_ref, xf_ref, dinv_ref):
    xb = x_ref[...]
    dinv = lax.rsqrt(deg_ref[...] + 1.0)
    dinv_ref[...] = dinv
    g1_ref[...] = jnp.dot(xb, w1_ref[...], preferred_element_type=jnp.float32) * dinv
    t = jnp.maximum(
        jnp.dot(xb, wf1_ref[...], preferred_element_type=jnp.float32) + bf1_ref[...],
        0.0,
    )
    xf_ref[...] = jnp.dot(t, wf2_ref[...], preferred_element_type=jnp.float32) + bf2_ref[...]


_tc_a = pl.pallas_call(
    _tc_a_body,
    grid=(GRID,),
    in_specs=[_row, _col, _wgt, _wgt, _bias, _wgt, _bias],
    out_specs=[_row, _row, _col],
    out_shape=[
        jax.ShapeDtypeStruct((NPAD, D), jnp.float32),
        jax.ShapeDtypeStruct((NPAD, D), jnp.float32),
        jax.ShapeDtypeStruct((NPAD, 1), jnp.float32),
    ],
)


def _tc_b_body(s_ref, dinv_ref, b1_ref, w2_ref, g2_ref):
    dinv = dinv_ref[...]
    pre = s_ref[...] * dinv + b1_ref[...]
    out1 = jnp.maximum(pre, 0.0)
    g2_ref[...] = jnp.dot(out1, w2_ref[...], preferred_element_type=jnp.float32) * dinv


_tc_b = pl.pallas_call(
    _tc_b_body,
    grid=(GRID,),
    in_specs=[_row, _col, _bias, _wgt],
    out_specs=_row,
    out_shape=jax.ShapeDtypeStruct((NPAD, D), jnp.float32),
)


def _tc_c_body(s_ref, dinv_ref, b2_ref, xf_ref, z_ref):
    pre = s_ref[...] * dinv_ref[...] + b2_ref[...]
    z_ref[...] = 0.5 * pre + 0.5 * xf_ref[...]


_tc_c = pl.pallas_call(
    _tc_c_body,
    grid=(GRID,),
    in_specs=[_row, _col, _bias, _row],
    out_specs=_row,
    out_shape=jax.ShapeDtypeStruct((NPAD, D), jnp.float32),
)


def kernel(x, edge_index, edge_label_index, W1, b1, W2, b2, Wf1, bf1, Wf2, bf2):
    ei = edge_index.astype(jnp.int32)
    eli = edge_label_index.astype(jnp.int32)
    src = ei[0]
    dst = ei[1]

    xp = jnp.pad(x, ((0, NPAD - N), (0, 0)))
    src3 = jnp.stack([src, src + NPAD]).reshape(NC, E_CHUNKS2, ECHUNK)
    dst3 = dst.reshape(E_CHUNKS2, ECHUNK)
    deg = _deg_kernel(dst3).reshape(NPAD, 1)

    b1r = b1.reshape(1, D)
    b2r = b2.reshape(1, D)
    bf1r = bf1.reshape(1, D)
    bf2r = bf2.reshape(1, D)

    g1, xf, dinv = _tc_a(xp, deg, W1, Wf1, bf1r, Wf2, bf2r)
    g1f = jnp.concatenate([g1[:, :DH], g1[:, DH:]], axis=0)
    s1h = _agg_kernel(g1f, src3, dst3)
    s1 = s1h.transpose(1, 0, 2).reshape(NPAD, D)
    g2 = _tc_b(s1, dinv, b1r, W2)
    g2f = jnp.concatenate([g2[:, :DH], g2[:, DH:]], axis=0)
    s2h = _agg_kernel(g2f, src3, dst3)
    s2 = s2h.transpose(1, 0, 2).reshape(NPAD, D)
    z = _tc_c(s2, dinv, b2r, xf)
    npad_lbl = NC * NS * L_ITER * LCHUNK - LBL
    ea3 = jnp.pad(eli[0], (0, npad_lbl)).reshape(NC * NS * L_ITER, LCHUNK)
    eb3 = jnp.pad(eli[1], (0, npad_lbl)).reshape(NC * NS * L_ITER, LCHUNK)
    out = _decode_kernel(z, ea3, eb3)
    return out
